# baseline (speedup 1.0000x reference)
import functools

import jax
import jax.numpy as jnp
from jax import lax
from jax.experimental import pallas as pl
from jax.experimental.pallas import tpu as pltpu
from jax.experimental.pallas import tpu_sc as plsc

_V = 1000000
_D = 64
_B, _L = 4096, 200
_N = _B * _L

_NC = 2
_NS = 16
_NW = _NC * _NS
_PER_W = _N // _NW
_IDXW = 128
_C = 256
_NCHUNK = _PER_W // _C

_mesh = plsc.VectorSubcoreMesh(core_axis_name="c", subcore_axis_name="s")


def _splat(val):
    return jnp.full((16,), val, jnp.int32)


@functools.partial(
    pl.kernel,
    out_type=jax.ShapeDtypeStruct((_B, _L, _D), jnp.float32),
    mesh=_mesh,
    compiler_params=pltpu.CompilerParams(needs_layout_passes=False),
    scratch_types=[
        pltpu.VMEM((_C,), jnp.int32),
        pltpu.VMEM((_C,), jnp.int32),
        pltpu.VMEM((_C, 2 * _D), jnp.float32),
        pltpu.VMEM((_C, _D), jnp.float32),
        pltpu.SemaphoreType.DMA,
    ],
)
def _emb_lookup(packed, idx_hbm, out_hbm, idx_v, idxj_v, rows_v, rows_c, sem):
    out2 = out_hbm.reshape(_N, _D)
    wid = lax.axis_index("s") * _NC + lax.axis_index("c")
    base = wid * _PER_W
    iota = lax.iota(jnp.int32, 16)
    one = _splat(1)
    half = _splat(_D)

    def chunk_body(i, carry):
        off = base + i * _C
        pltpu.sync_copy(idx_hbm.at[pl.ds(off, _C)], idx_v)
        for m in range(_C // 16):
            v = idx_v[pl.ds(16 * m, 16)]
            idxj_v[pl.ds(16 * m, 16)] = lax.shift_right_logical(v, one)
        copies = []
        for j in range(_C // _IDXW):
            copies.append(
                pltpu.async_copy(
                    packed.at[idxj_v.at[pl.ds(j * _IDXW, _IDXW)]],
                    rows_v.at[pl.ds(j * _IDXW, _IDXW)],
                    sem,
                )
            )
        for c in copies:
            c.wait()

        def grp_body(g, carry2):
            yv = idx_v[pl.ds(g * 16, 16)]
            rowvec = g * 16 + iota
            colbase = (yv & one) * half
            for c in range(_D):
                vals = plsc.load_gather(rows_v, [rowvec, colbase + _splat(c)])
                plsc.store_scatter(rows_c, [rowvec, _splat(c)], vals)
            return carry2

        lax.fori_loop(0, _C // 16, grp_body, 0)
        pltpu.sync_copy(rows_c, out2.at[pl.ds(off, _C)])
        return carry

    lax.fori_loop(0, _NCHUNK, chunk_body, 0)


def kernel(y, table):
    packed = table.reshape(_V // 2, 2 * _D)
    idx = y.reshape(_N)
    return _emb_lookup(packed, idx)


# speedup vs baseline: 1.8429x; 1.8429x over previous
import functools

import jax
import jax.numpy as jnp
from jax import lax
from jax.experimental import pallas as pl
from jax.experimental.pallas import tpu as pltpu
from jax.experimental.pallas import tpu_sc as plsc

_V = 1000000
_D = 64
_B, _L = 4096, 200
_N = _B * _L

_NC = 2
_NS = 16
_NW = _NC * _NS
_PER_W = _N // _NW
_IDXW = 128
_C = 256
_NCHUNK = _PER_W // _C

_mesh = plsc.VectorSubcoreMesh(core_axis_name="c", subcore_axis_name="s")


@functools.partial(
    pl.kernel,
    out_type=jax.ShapeDtypeStruct((_B, _L, _D), jnp.float32),
    mesh=_mesh,
    compiler_params=pltpu.CompilerParams(needs_layout_passes=False),
    scratch_types=[
        pltpu.VMEM((_C,), jnp.int32),
        pltpu.VMEM((_C,), jnp.int32),
        pltpu.VMEM((_C, 2 * _D), jnp.float32),
        pltpu.VMEM((_C, _D), jnp.float32),
        pltpu.SemaphoreType.DMA,
    ],
)
def _emb_lookup(packed, idxj_hbm, poff_hbm, out_hbm, idxj_v, poff_v,
                rows_v, rows_c, sem):
    out2 = out_hbm.reshape(_N, _D)
    wid = lax.axis_index("s") * _NC + lax.axis_index("c")
    base = wid * _PER_W

    def chunk_body(i, carry):
        off = base + i * _C
        pltpu.sync_copy(idxj_hbm.at[pl.ds(off, _C)], idxj_v)
        pltpu.sync_copy(poff_hbm.at[pl.ds(off, _C)], poff_v)
        copies = []
        for j in range(_C // _IDXW):
            copies.append(
                pltpu.async_copy(
                    packed.at[idxj_v.at[pl.ds(j * _IDXW, _IDXW)]],
                    rows_v.at[pl.ds(j * _IDXW, _IDXW)],
                    sem,
                )
            )
        for c in copies:
            c.wait()

        def grp_body(g, carry2):
            poff16 = poff_v[pl.ds(16 * g, 16)]
            for l in range(16):
                r = 16 * g + l
                p = poff16[l]
                for k in range(_D // 16):
                    rows_c[r, pl.ds(16 * k, 16)] = (
                        rows_v[r, pl.ds(p + 16 * k, 16)])
            return carry2

        lax.fori_loop(0, _C // 16, grp_body, 0)
        pltpu.sync_copy(rows_c, out2.at[pl.ds(off, _C)])
        return carry

    lax.fori_loop(0, _NCHUNK, chunk_body, 0)


def kernel(y, table):
    packed = table.reshape(_V // 2, 2 * _D)
    yf = y.reshape(_N)
    idxj = yf >> 1
    poff = (yf & 1) * _D
    return _emb_lookup(packed, idxj, poff)
